# baseline (speedup 1.0000x reference)
import jax
import jax.numpy as jnp
from jax.experimental import pallas as pl
from jax.experimental.pallas import tpu as pltpu

_CH = 16
_EMB = 128
_HB = 16
_W = 384


def _fused_kernel(xp, xc, xn, fm, mle, mre, w1, b1, w2, b2, w3, b3, out):
    w = _W
    n = _HB * w
    xe = jnp.concatenate(
        [xp[0][:, n - 2 * w:], xc[0], xn[0][:, :2 * w]], axis=1)

    up = xe[:, w:w + n]
    mid = xe[:, 2 * w:2 * w + n]
    dn = xe[:, 3 * w:3 * w + n]
    ul = xe[:, w + 1:w + 1 + n]
    ur = xe[:, w - 1:w - 1 + n]
    cl = xe[:, 2 * w + 1:2 * w + 1 + n]
    cr = xe[:, 2 * w - 1:2 * w - 1 + n]
    dl = xe[:, 3 * w + 1:3 * w + 1 + n]
    dr = xe[:, 3 * w - 1:3 * w - 1 + n]
    ml = mle[...]
    mr = mre[...]

    gx = (ul + 2.0 * cl + dl) * ml - (ur + 2.0 * cr + dr) * mr
    gy = (dl - ul) * ml + (dr - ur) * mr + 2.0 * (dn - up)

    xa = xe[3:4, :]
    pmf = jnp.maximum(jnp.maximum(xa[:, :n + 2 * w], xa[:, w:n + 3 * w]),
                      xa[:, 2 * w:])
    pooled = jnp.maximum(
        jnp.maximum(pmf[:, w + 1:w + 1 + n] * ml, pmf[:, w - 1:w - 1 + n] * mr),
        pmf[:, w:w + n])
    act = jnp.where((pooled > 0.1) & (fm[0] != 0), 1.0, 0.0)

    f = jnp.concatenate([mid, gx, gy], axis=0)
    prec = jax.lax.Precision.DEFAULT
    h1 = jnp.maximum(
        jnp.dot(w1[...], f, preferred_element_type=jnp.float32, precision=prec)
        + b1[...], 0.0)
    h2 = jnp.maximum(
        jnp.dot(w2[...], h1, preferred_element_type=jnp.float32, precision=prec)
        + b2[...], 0.0)
    u = (jnp.dot(w3[...], h2, preferred_element_type=jnp.float32, precision=prec)
         + b3[...])
    out[0] = u * act


def kernel(x, fire_mask, W1, b1, W2, b2, W3, b3):
    B, C, H, W = x.shape
    nh = H // _HB
    n = _HB * W
    xpad = jnp.pad(x, ((0, 0), (0, 0), (_HB, _HB), (0, 0)))
    xflat = xpad.reshape(B, C, (H + 2 * _HB) * W)
    fmflat = fire_mask.reshape(B, 1, H * W)
    wcol = jnp.arange(n, dtype=jnp.int32) % W
    mle = (wcol != W - 1).astype(jnp.float32).reshape(1, n)
    mre = (wcol != 0).astype(jnp.float32).reshape(1, n)
    b1c = b1.reshape(_EMB, 1)
    b2c = b2.reshape(_EMB, 1)
    b3c = b3.reshape(_CH, 1)

    def spec_x(off):
        return pl.BlockSpec((1, C, n), lambda b, h: (b, 0, h + off))

    outf = pl.pallas_call(
        _fused_kernel,
        grid=(B, nh),
        in_specs=[
            spec_x(0), spec_x(1), spec_x(2),
            pl.BlockSpec((1, 1, n), lambda b, h: (b, 0, h)),
            pl.BlockSpec((1, n), lambda b, h: (0, 0)),
            pl.BlockSpec((1, n), lambda b, h: (0, 0)),
            pl.BlockSpec((_EMB, 3 * _CH), lambda b, h: (0, 0)),
            pl.BlockSpec((_EMB, 1), lambda b, h: (0, 0)),
            pl.BlockSpec((_EMB, _EMB), lambda b, h: (0, 0)),
            pl.BlockSpec((_EMB, 1), lambda b, h: (0, 0)),
            pl.BlockSpec((_CH, _EMB), lambda b, h: (0, 0)),
            pl.BlockSpec((_CH, 1), lambda b, h: (0, 0)),
        ],
        out_specs=pl.BlockSpec((1, C, n), lambda b, h: (b, 0, h)),
        out_shape=jax.ShapeDtypeStruct((B, C, H * W), jnp.float32),
        compiler_params=pltpu.CompilerParams(
            dimension_semantics=("parallel", "parallel")),
    )(xflat, xflat, xflat, fmflat, mle, mre, W1, b1c, W2, b2c, W3, b3c)
    return outf.reshape(B, C, H, W)


# speedup vs baseline: 1.0804x; 1.0804x over previous
import jax
import jax.numpy as jnp
from jax.experimental import pallas as pl
from jax.experimental.pallas import tpu as pltpu

_CH = 16
_EMB = 128
_HB = 16
_W = 384


def _fused_kernel(xt, xc, xb, fm, mle, mre, w1, b1, w2, b2, w3, b3, out):
    w = _W
    n = _HB * w
    nh = pl.num_programs(1)
    i = pl.program_id(1)
    top = xt[0] * jnp.where(i > 0, 1.0, 0.0)
    bot = xb[0] * jnp.where(i < nh - 1, 1.0, 0.0)
    xe = jnp.concatenate([top, xc[0], bot], axis=1)

    up = xe[:, w:w + n]
    mid = xe[:, 2 * w:2 * w + n]
    dn = xe[:, 3 * w:3 * w + n]
    ul = xe[:, w + 1:w + 1 + n]
    ur = xe[:, w - 1:w - 1 + n]
    cl = xe[:, 2 * w + 1:2 * w + 1 + n]
    cr = xe[:, 2 * w - 1:2 * w - 1 + n]
    dl = xe[:, 3 * w + 1:3 * w + 1 + n]
    dr = xe[:, 3 * w - 1:3 * w - 1 + n]
    ml = mle[...]
    mr = mre[...]

    gx = (ul + 2.0 * cl + dl) * ml - (ur + 2.0 * cr + dr) * mr
    gy = (dl - ul) * ml + (dr - ur) * mr + 2.0 * (dn - up)

    xa = xe[3:4, :]
    pmf = jnp.maximum(jnp.maximum(xa[:, :n + 2 * w], xa[:, w:n + 3 * w]),
                      xa[:, 2 * w:])
    pooled = jnp.maximum(
        jnp.maximum(pmf[:, w + 1:w + 1 + n] * ml, pmf[:, w - 1:w - 1 + n] * mr),
        pmf[:, w:w + n])
    act = jnp.where((pooled > 0.1) & (fm[0] != 0), 1.0, 0.0)

    f = jnp.concatenate([mid, gx, gy], axis=0)
    prec = jax.lax.Precision.DEFAULT
    h1 = jnp.maximum(
        jnp.dot(w1[...], f, preferred_element_type=jnp.float32, precision=prec)
        + b1[...], 0.0)
    h2 = jnp.maximum(
        jnp.dot(w2[...], h1, preferred_element_type=jnp.float32, precision=prec)
        + b2[...], 0.0)
    u = (jnp.dot(w3[...], h2, preferred_element_type=jnp.float32, precision=prec)
         + b3[...])
    out[0] = u * act


def kernel(x, fire_mask, W1, b1, W2, b2, W3, b3):
    B, C, H, W = x.shape
    nh = H // _HB
    n = _HB * W
    k = _HB // 2
    nhb = H * W // (2 * W)
    xflat = x.reshape(B, C, H * W)
    fmflat = fire_mask.reshape(B, 1, H * W)
    wcol = jnp.arange(n, dtype=jnp.int32) % W
    mle = (wcol != W - 1).astype(jnp.float32).reshape(1, n)
    mre = (wcol != 0).astype(jnp.float32).reshape(1, n)
    b1c = b1.reshape(_EMB, 1)
    b2c = b2.reshape(_EMB, 1)
    b3c = b3.reshape(_CH, 1)

    outf = pl.pallas_call(
        _fused_kernel,
        grid=(B, nh),
        in_specs=[
            pl.BlockSpec((1, C, 2 * W),
                         lambda b, h: (b, 0, jnp.maximum(k * h - 1, 0))),
            pl.BlockSpec((1, C, n), lambda b, h: (b, 0, h)),
            pl.BlockSpec((1, C, 2 * W),
                         lambda b, h: (b, 0, jnp.minimum(k * (h + 1), nhb - 1))),
            pl.BlockSpec((1, 1, n), lambda b, h: (b, 0, h)),
            pl.BlockSpec((1, n), lambda b, h: (0, 0)),
            pl.BlockSpec((1, n), lambda b, h: (0, 0)),
            pl.BlockSpec((_EMB, 3 * _CH), lambda b, h: (0, 0)),
            pl.BlockSpec((_EMB, 1), lambda b, h: (0, 0)),
            pl.BlockSpec((_EMB, _EMB), lambda b, h: (0, 0)),
            pl.BlockSpec((_EMB, 1), lambda b, h: (0, 0)),
            pl.BlockSpec((_CH, _EMB), lambda b, h: (0, 0)),
            pl.BlockSpec((_CH, 1), lambda b, h: (0, 0)),
        ],
        out_specs=pl.BlockSpec((1, C, n), lambda b, h: (b, 0, h)),
        out_shape=jax.ShapeDtypeStruct((B, C, H * W), jnp.float32),
        compiler_params=pltpu.CompilerParams(
            dimension_semantics=("parallel", "arbitrary")),
    )(xflat, xflat, xflat, fmflat, mle, mre, W1, b1c, W2, b2c, W3, b3c)
    return outf.reshape(B, C, H, W)
